# Initial kernel scaffold; baseline (speedup 1.0000x reference)
import jax
import jax.numpy as jnp
from jax.experimental import pallas as pl


def kernel(x, edge_index, W1, b1, W2, b2):
    raise NotImplementedError("write your pallas kernel here")


# speedup vs baseline: 10.8335x; 10.8335x over previous
import functools

import jax
import jax.numpy as jnp
from jax import lax
from jax.experimental import pallas as pl
from jax.experimental.pallas import tpu as pltpu
from jax.experimental.pallas import tpu_sc as plsc

N = 10000
D = 128
E = 320000
NC = 2
NS = 16
NW = NC * NS
EPW = E // NW
CHUNK = 128
NCHUNK = -(-EPW // CHUNK)
EPAD = NCHUNK * CHUNK
NA = N + 8
RPT = N // NS
RZ = 125

_mesh = plsc.VectorSubcoreMesh(
    core_axis_name="c", subcore_axis_name="s", num_cores=NC, num_subcores=NS
)


@functools.partial(
    pl.kernel,
    out_type=jax.ShapeDtypeStruct((NC, NS, RPT, D), jnp.float32),
    mesh=_mesh,
    scratch_types=[
        pltpu.VMEM((CHUNK,), jnp.int32),
        pltpu.VMEM((CHUNK, D), jnp.float32),
        pltpu.VMEM_SHARED((NA, D), jnp.float32),
    ],
)
def _sc_degree(dst_hbm, out_hbm, idx_v, ones_v, acc):
    c = lax.axis_index("c")
    s = lax.axis_index("s")
    wid = c * NS + s

    def zrow(i, _):
        for k in range(D // 16):
            ones_v[i, pl.ds(k * 16, 16)] = jnp.zeros((16,), jnp.float32)
        return 0

    lax.fori_loop(0, CHUNK, zrow, 0)
    for r in range(RPT // RZ):
        pltpu.sync_copy(
            ones_v.at[pl.ds(0, RZ)], acc.at[pl.ds(s * RPT + r * RZ, RZ)]
        )

    def orow(i, _):
        for k in range(D // 16):
            ones_v[i, pl.ds(k * 16, 16)] = jnp.ones((16,), jnp.float32)
        return 0

    lax.fori_loop(0, CHUNK, orow, 0)
    plsc.subcore_barrier()

    def chunk(j, _):
        pltpu.sync_copy(dst_hbm.at[wid, j], idx_v)
        pltpu.sync_copy(ones_v, acc.at[idx_v], add=True)
        return 0

    lax.fori_loop(0, NCHUNK, chunk, 0)
    plsc.subcore_barrier()
    pltpu.sync_copy(acc.at[pl.ds(s * RPT, RPT)], out_hbm.at[c, s])


@functools.partial(
    pl.kernel,
    out_type=jax.ShapeDtypeStruct((NC, NS, RPT, D), jnp.float32),
    mesh=_mesh,
    scratch_types=[
        pltpu.VMEM((CHUNK,), jnp.int32),
        pltpu.VMEM((CHUNK,), jnp.int32),
        pltpu.VMEM((CHUNK, D), jnp.float32),
        pltpu.VMEM_SHARED((NA, D), jnp.float32),
        pltpu.SemaphoreType.DMA,
    ],
)
def _sc_aggregate(
    table_hbm, src_hbm, dst_hbm, out_hbm, src_v, dst_v, rows_v, acc, sem
):
    c = lax.axis_index("c")
    s = lax.axis_index("s")
    wid = c * NS + s

    def zrow(i, _):
        for k in range(D // 16):
            rows_v[i, pl.ds(k * 16, 16)] = jnp.zeros((16,), jnp.float32)
        return 0

    lax.fori_loop(0, CHUNK, zrow, 0)
    for r in range(RPT // RZ):
        pltpu.sync_copy(
            rows_v.at[pl.ds(0, RZ)], acc.at[pl.ds(s * RPT + r * RZ, RZ)]
        )
    plsc.subcore_barrier()

    def chunk(j, _):
        pltpu.sync_copy(src_hbm.at[wid, j], src_v)
        pltpu.sync_copy(dst_hbm.at[wid, j], dst_v)
        pltpu.async_copy(table_hbm.at[src_v], rows_v, sem).wait()
        pltpu.sync_copy(rows_v, acc.at[dst_v], add=True)
        return 0

    lax.fori_loop(0, NCHUNK, chunk, 0)
    plsc.subcore_barrier()
    pltpu.sync_copy(acc.at[pl.ds(s * RPT, RPT)], out_hbm.at[c, s])


def _tc_matmul(x_ref, w_ref, out_ref):
    out_ref[...] = jnp.dot(
        x_ref[...], w_ref[...], preferred_element_type=jnp.float32
    )


def _tc_scale(hist_ref, h_ref, out_ref):
    dinv = lax.rsqrt(1.0 + hist_ref[0] + hist_ref[1])
    out_ref[...] = dinv * h_ref[...]


def _tc_mid(hist_ref, agg_ref, hs_ref, b_ref, w_ref, out_ref):
    dinv = lax.rsqrt(1.0 + hist_ref[0] + hist_ref[1])
    pre = dinv * (agg_ref[0] + agg_ref[1] + hs_ref[...]) + b_ref[...]
    h1 = jnp.maximum(pre, 0.0)
    out_ref[...] = dinv * jnp.dot(
        h1, w_ref[...], preferred_element_type=jnp.float32
    )


def _tc_final(hist_ref, agg_ref, hs_ref, b_ref, out_ref):
    dinv = lax.rsqrt(1.0 + hist_ref[0] + hist_ref[1])
    out_ref[...] = dinv * (agg_ref[0] + agg_ref[1] + hs_ref[...]) + b_ref[...]


_f32 = functools.partial(jax.ShapeDtypeStruct, dtype=jnp.float32)


@jax.jit
def kernel(x, edge_index, W1, b1, W2, b2):
    pad_src = jnp.zeros((NW, EPAD - EPW), jnp.int32)
    pad_dst = jnp.full((NW, EPAD - EPW), N, jnp.int32)
    src = jnp.concatenate(
        [edge_index[0].reshape(NW, EPW), pad_src], axis=1
    ).reshape(NW, NCHUNK, CHUNK)
    dst = jnp.concatenate(
        [edge_index[1].reshape(NW, EPW), pad_dst], axis=1
    ).reshape(NW, NCHUNK, CHUNK)

    hist = _sc_degree(dst).reshape(NC, N, D)[:, :, 0:1]
    h1 = pl.pallas_call(_tc_matmul, out_shape=_f32((N, D)))(x, W1)

    hs1 = pl.pallas_call(_tc_scale, out_shape=_f32((N, D)))(hist, h1)

    agg1 = _sc_aggregate(hs1, src, dst).reshape(NC, N, D)

    hs2 = pl.pallas_call(_tc_mid, out_shape=_f32((N, D)))(
        hist, agg1, hs1, b1.reshape(1, D), W2
    )

    agg2 = _sc_aggregate(hs2, src, dst).reshape(NC, N, D)

    out = pl.pallas_call(_tc_final, out_shape=_f32((N, D)))(
        hist, agg2, hs2, b2.reshape(1, D)
    )
    return out


# speedup vs baseline: 14.2924x; 1.3193x over previous
import functools

import jax
import jax.numpy as jnp
from jax import lax
from jax.experimental import pallas as pl
from jax.experimental.pallas import tpu as pltpu
from jax.experimental.pallas import tpu_sc as plsc

N = 10000
D = 128
E = 320000
NC = 2
NS = 16
NW = NC * NS
EPW = E // NW
CHUNK = 128
NCHUNK = -(-EPW // CHUNK)
EPAD = NCHUNK * CHUNK
NA = N + 8
RPT = N // NS
RZ = 125

_mesh = plsc.VectorSubcoreMesh(
    core_axis_name="c", subcore_axis_name="s", num_cores=NC, num_subcores=NS
)


@functools.partial(
    pl.kernel,
    out_type=jax.ShapeDtypeStruct((NC, NS, RPT, D), jnp.float32),
    mesh=_mesh,
    scratch_types=[
        pltpu.VMEM((CHUNK,), jnp.int32),
        pltpu.VMEM((CHUNK, D), jnp.float32),
        pltpu.VMEM_SHARED((NA, D), jnp.float32),
    ],
)
def _sc_degree(dst_hbm, out_hbm, idx_v, ones_v, acc):
    c = lax.axis_index("c")
    s = lax.axis_index("s")
    wid = c * NS + s

    def zrow(i, _):
        for k in range(D // 16):
            ones_v[i, pl.ds(k * 16, 16)] = jnp.zeros((16,), jnp.float32)
        return 0

    lax.fori_loop(0, CHUNK, zrow, 0)
    for r in range(RPT // RZ):
        pltpu.sync_copy(
            ones_v.at[pl.ds(0, RZ)], acc.at[pl.ds(s * RPT + r * RZ, RZ)]
        )

    def orow(i, _):
        for k in range(D // 16):
            ones_v[i, pl.ds(k * 16, 16)] = jnp.ones((16,), jnp.float32)
        return 0

    lax.fori_loop(0, CHUNK, orow, 0)
    plsc.subcore_barrier()

    def chunk(j, _):
        pltpu.sync_copy(dst_hbm.at[wid, j, 1], idx_v)
        pltpu.sync_copy(ones_v, acc.at[idx_v], add=True)
        return 0

    lax.fori_loop(0, NCHUNK, chunk, 0)
    plsc.subcore_barrier()
    pltpu.sync_copy(acc.at[pl.ds(s * RPT, RPT)], out_hbm.at[c, s])


@functools.partial(
    pl.kernel,
    out_type=jax.ShapeDtypeStruct((NC, NS, RPT, D), jnp.float32),
    mesh=_mesh,
    scratch_types=[
        pltpu.VMEM((2, CHUNK), jnp.int32),
        pltpu.VMEM((2, CHUNK), jnp.int32),
        pltpu.VMEM((CHUNK, D), jnp.float32),
        pltpu.VMEM((CHUNK, D), jnp.float32),
        pltpu.VMEM_SHARED((NA, D), jnp.float32),
        pltpu.SemaphoreType.DMA,
        pltpu.SemaphoreType.DMA,
    ],
)
def _sc_aggregate(
    table_hbm, idx_hbm, out_hbm, idx_a, idx_b, rows_a, rows_b, acc, sem_a,
    sem_b,
):
    c = lax.axis_index("c")
    s = lax.axis_index("s")
    wid = c * NS + s

    def zrow(i, _):
        for k in range(D // 16):
            rows_a[i, pl.ds(k * 16, 16)] = jnp.zeros((16,), jnp.float32)
        return 0

    lax.fori_loop(0, CHUNK, zrow, 0)
    for r in range(RPT // RZ):
        pltpu.sync_copy(
            rows_a.at[pl.ds(0, RZ)], acc.at[pl.ds(s * RPT + r * RZ, RZ)]
        )
    plsc.subcore_barrier()

    pltpu.sync_copy(idx_hbm.at[wid, 0], idx_a)
    pltpu.async_copy(table_hbm.at[idx_a.at[0]], rows_a, sem_a)

    def step(jj, _):
        j1 = 2 * jj + 1
        pltpu.sync_copy(idx_hbm.at[wid, j1], idx_b)
        pltpu.async_copy(table_hbm.at[idx_b.at[0]], rows_b, sem_b)
        pltpu.make_async_copy(table_hbm.at[idx_a.at[0]], rows_a, sem_a).wait()
        pltpu.sync_copy(rows_a, acc.at[idx_a.at[1]], add=True)
        pltpu.sync_copy(idx_hbm.at[wid, j1 + 1], idx_a)
        pltpu.async_copy(table_hbm.at[idx_a.at[0]], rows_a, sem_a)
        pltpu.make_async_copy(table_hbm.at[idx_b.at[0]], rows_b, sem_b).wait()
        pltpu.sync_copy(rows_b, acc.at[idx_b.at[1]], add=True)
        return 0

    lax.fori_loop(0, (NCHUNK - 1) // 2, step, 0)
    pltpu.make_async_copy(table_hbm.at[idx_a.at[0]], rows_a, sem_a).wait()
    pltpu.sync_copy(rows_a, acc.at[idx_a.at[1]], add=True)
    plsc.subcore_barrier()
    pltpu.sync_copy(acc.at[pl.ds(s * RPT, RPT)], out_hbm.at[c, s])


def _tc_matmul(x_ref, w_ref, out_ref):
    out_ref[...] = jnp.dot(
        x_ref[...], w_ref[...], preferred_element_type=jnp.float32
    )


def _tc_scale(hist_ref, h_ref, out_ref):
    dinv = lax.rsqrt(1.0 + hist_ref[0] + hist_ref[1])
    out_ref[...] = dinv * h_ref[...]


def _tc_mid(hist_ref, agg_ref, hs_ref, b_ref, w_ref, out_ref):
    dinv = lax.rsqrt(1.0 + hist_ref[0] + hist_ref[1])
    pre = dinv * (agg_ref[0] + agg_ref[1] + hs_ref[...]) + b_ref[...]
    h1 = jnp.maximum(pre, 0.0)
    out_ref[...] = dinv * jnp.dot(
        h1, w_ref[...], preferred_element_type=jnp.float32
    )


def _tc_final(hist_ref, agg_ref, hs_ref, b_ref, out_ref):
    dinv = lax.rsqrt(1.0 + hist_ref[0] + hist_ref[1])
    out_ref[...] = dinv * (agg_ref[0] + agg_ref[1] + hs_ref[...]) + b_ref[...]


_f32 = functools.partial(jax.ShapeDtypeStruct, dtype=jnp.float32)


@jax.jit
def kernel(x, edge_index, W1, b1, W2, b2):
    pad_src = jnp.zeros((NW, EPAD - EPW), jnp.int32)
    pad_dst = jnp.full((NW, EPAD - EPW), N, jnp.int32)
    src = jnp.concatenate(
        [edge_index[0].reshape(NW, EPW), pad_src], axis=1
    ).reshape(NW, NCHUNK, 1, CHUNK)
    dst = jnp.concatenate(
        [edge_index[1].reshape(NW, EPW), pad_dst], axis=1
    ).reshape(NW, NCHUNK, 1, CHUNK)
    idx = jnp.concatenate([src, dst], axis=2)

    hist = _sc_degree(idx).reshape(NC, N, D)[:, :, 0:1]
    h1 = pl.pallas_call(_tc_matmul, out_shape=_f32((N, D)))(x, W1)

    hs1 = pl.pallas_call(_tc_scale, out_shape=_f32((N, D)))(hist, h1)

    agg1 = _sc_aggregate(hs1, idx).reshape(NC, N, D)

    hs2 = pl.pallas_call(_tc_mid, out_shape=_f32((N, D)))(
        hist, agg1, hs1, b1.reshape(1, D), W2
    )

    agg2 = _sc_aggregate(hs2, idx).reshape(NC, N, D)

    out = pl.pallas_call(_tc_final, out_shape=_f32((N, D)))(
        hist, agg2, hs2, b2.reshape(1, D)
    )
    return out
